# baseline (speedup 1.0000x reference)
import jax
import jax.numpy as jnp
from jax.experimental import pallas as pl
from jax.experimental.pallas import tpu as pltpu

_B, _H, _W = 16, 512, 512
_N = _B * _H * _W
_TOPK = int(_N * 0.1)
_ROWS = _N // _W
_BLK = 512
_NBLK = _ROWS // _BLK

_NTHR_C = 4
_NPASS_C = 8
_NTHR_F = 8
_NPH = 1 + _NPASS_C + 1


_SOBX = jnp.array([[-1.0, 0.0, 1.0], [-2.0, 0.0, 2.0], [-1.0, 0.0, 1.0]],
                  dtype=jnp.float32).reshape(1, 1, 3, 3)
_SOBY = jnp.array([[-1.0, -2.0, -1.0], [0.0, 0.0, 0.0], [1.0, 2.0, 1.0]],
                  dtype=jnp.float32).reshape(1, 1, 3, 3)


def _edges_conv(img):
    dn = ("NCHW", "OIHW", "NCHW")
    ex = jax.lax.conv_general_dilated(img, _SOBX, window_strides=(1, 1),
                                      padding=((1, 1), (1, 1)),
                                      dimension_numbers=dn)
    ey = jax.lax.conv_general_dilated(img, _SOBY, window_strides=(1, 1),
                                      padding=((1, 1), (1, 1)),
                                      dimension_numbers=dn)
    return jnp.sqrt(ex ** 2 + ey ** 2 + 1e-06)


def _lanesum(x):
    return jnp.sum(x.reshape(_BLK // 8, 8, _W), axis=0)


def _loss_kernel(p_ref, t_ref, out_ref,
                 e_p, e_t, acc, cacc, sacc, vacc, brk, res, mean_s):
    ph = pl.program_id(0)
    b = pl.program_id(1)
    nk = jnp.float32(_TOPK)

    @pl.when((ph == 0) & (b == 0))
    def _():
        acc[...] = jnp.zeros_like(acc)
        for i in range(2):
            brk[i, 0] = 0
            brk[i, 1] = 0x7F7FFFFF

    @pl.when(ph == 0)
    def _():
        ep = p_ref[...]
        et = t_ref[...]
        e_p[pl.ds(b * _BLK, _BLK), :] = ep
        e_t[pl.ds(b * _BLK, _BLK), :] = et
        acc[0] += _lanesum(ep)
        acc[1] += _lanesum(ep * ep)
        acc[2] += _lanesum(et)
        acc[3] += _lanesum(et * et)

        @pl.when(b == _NBLK - 1)
        def _():
            mean_s[0] = jnp.sum(acc[0]) / jnp.float32(_N)
            mean_s[1] = jnp.sum(acc[2]) / jnp.float32(_N)

    def thresholds(i, nthr, include_hi):
        lo = brk[i, 0]
        hi = brk[i, 1]
        if include_hi:
            step = (hi - lo) // nthr
            us = [lo + step * j for j in range(1, nthr)] + [hi]
        else:
            step = (hi - lo) // (nthr + 1)
            us = [lo + step * j for j in range(1, nthr + 1)]
        return us, [jax.lax.bitcast_convert_type(u, jnp.float32) for u in us]

    def bracket_update(i, nthr, cnts, us):
        lo, hi = brk[i, 0], brk[i, 1]
        new_lo, new_hi = lo, hi
        for j in range(nthr - 1, -1, -1):
            ge = cnts[j] >= nk
            new_lo = jnp.where(ge, jnp.maximum(new_lo, us[j]), new_lo)
            new_hi = jnp.where(ge, new_hi, us[j])
        brk[i, 0] = new_lo
        brk[i, 1] = new_hi
        return new_lo, new_hi

    @pl.when((ph >= 1) & (ph <= _NPASS_C))
    def _():
        @pl.when(b == 0)
        def _():
            cacc[...] = jnp.zeros_like(cacc)

        for i, e in enumerate((e_p, e_t)):
            x = e[pl.ds(b * _BLK, _BLK), :]
            _, thrs = thresholds(i, _NTHR_C, include_hi=False)
            for j in range(_NTHR_C):
                cacc[i, j] += _lanesum((x > thrs[j]).astype(jnp.float32))

        @pl.when(b == _NBLK - 1)
        def _():
            for i in range(2):
                us, _ = thresholds(i, _NTHR_C, include_hi=False)
                cnts = [jnp.sum(cacc[i, j]) for j in range(_NTHR_C)]
                bracket_update(i, _NTHR_C, cnts, us)

    @pl.when(ph == 1)
    def _():
        @pl.when(b == 0)
        def _():
            vacc[...] = jnp.zeros_like(vacc)

        for i, e in enumerate((e_p, e_t)):
            x = e[pl.ds(b * _BLK, _BLK), :]
            d = x - mean_s[i]
            vacc[i] += _lanesum(d * d)

    @pl.when(ph == _NPH - 1)
    def _():
        @pl.when(b == 0)
        def _():
            cacc[...] = jnp.zeros_like(cacc)
            sacc[...] = jnp.zeros_like(sacc)

        for i, e in enumerate((e_p, e_t)):
            x = e[pl.ds(b * _BLK, _BLK), :]
            _, thrs = thresholds(i, _NTHR_F, include_hi=True)
            for j in range(_NTHR_F):
                mask = x > thrs[j]
                cacc[i, j] += _lanesum(mask.astype(jnp.float32))
                sacc[i, j] += _lanesum(jnp.where(mask, x - thrs[j], 0.0))

        @pl.when(b == _NBLK - 1)
        def _():
            for i in range(2):
                us, vs = thresholds(i, _NTHR_F, include_hi=True)
                cnts = [jnp.sum(cacc[i, j]) for j in range(_NTHR_F)]
                sms = [jnp.sum(sacc[i, j]) for j in range(_NTHR_F)]
                new_lo, new_hi = bracket_update(i, _NTHR_F, cnts, us)
                cg = cnts[_NTHR_F - 1]
                hg = sms[_NTHR_F - 1]
                for j in range(_NTHR_F - 2, -1, -1):
                    lt = cnts[j] < nk
                    cg = jnp.where(lt, cnts[j], cg)
                    hg = jnp.where(lt, sms[j], hg)
                v_lo = jax.lax.bitcast_convert_type(new_lo, jnp.float32)
                v_hi = jax.lax.bitcast_convert_type(new_hi, jnp.float32)
                mid = 0.5 * (v_lo + v_hi)
                res[i] = v_hi + (hg + (nk - cg) * (mid - v_hi)) / nk

            n_f = jnp.float32(_N)
            mean_p = mean_s[0]
            mean_t = mean_s[1]
            var_p = jnp.sum(vacc[0]) / (n_f - 1.0)
            var_t = jnp.sum(vacc[1]) / (n_f - 1.0)
            stats_loss = jnp.abs(mean_p - mean_t) + jnp.abs(
                jnp.sqrt(var_p) - jnp.sqrt(var_t))
            out_ref[0, 0] = stats_loss + jnp.abs(res[0] - res[1])


def kernel(pred, target, source):
    p = _edges_conv(pred).reshape(_ROWS, _W)
    t = _edges_conv(target).reshape(_ROWS, _W)

    def in_map(ph, b):
        return (jnp.where(ph == 0, b, _NBLK - 1), 0)

    out = pl.pallas_call(
        _loss_kernel,
        grid=(_NPH, _NBLK),
        in_specs=[
            pl.BlockSpec((_BLK, _W), in_map),
            pl.BlockSpec((_BLK, _W), in_map),
        ],
        out_specs=pl.BlockSpec(memory_space=pltpu.SMEM),
        out_shape=jax.ShapeDtypeStruct((1, 1), jnp.float32),
        scratch_shapes=[
            pltpu.VMEM((_ROWS, _W), jnp.float32),
            pltpu.VMEM((_ROWS, _W), jnp.float32),
            pltpu.VMEM((4, 8, _W), jnp.float32),
            pltpu.VMEM((2, _NTHR_F, 8, _W), jnp.float32),
            pltpu.VMEM((2, _NTHR_F, 8, _W), jnp.float32),
            pltpu.VMEM((2, 8, _W), jnp.float32),
            pltpu.SMEM((2, 2), jnp.int32),
            pltpu.SMEM((2,), jnp.float32),
            pltpu.SMEM((2,), jnp.float32),
        ],
    )(p, t)
    return out[0, 0]
